# baseline (speedup 1.0000x reference)
import functools

import jax
import jax.numpy as jnp
from jax.experimental import pallas as pl
from jax.experimental.pallas import tpu as pltpu

EPS = 1e-5
VMEM_LIMIT_BYTES = 48 * 1024 * 1024
L_TILE_MAX = 2048


def _round_up(x, m):
    return (x + m - 1) // m * m


def _stats_kernel(x_ref, w_ref, part_ref):
    y = jnp.dot(w_ref[...], x_ref[0], preferred_element_type=jnp.float32)
    part_ref[0, 0, :, 0:1] = jnp.sum(y, axis=1, keepdims=True)
    part_ref[0, 0, :, 1:2] = jnp.sum(y * y, axis=1, keepdims=True)


def _apply_kernel(x_ref, w_ref, gamma_ref, beta_ref, part_ref, o_ref, *, inv_n):
    part = part_ref[...]
    s = jnp.sum(part[:, :, :, 0:1], axis=(0, 1))
    sq = jnp.sum(part[:, :, :, 1:2], axis=(0, 1))
    mean = s * inv_n
    var = jnp.maximum(sq * inv_n - mean * mean, 0.0)
    inv_std = jax.lax.rsqrt(var + EPS)
    scale = gamma_ref[...] * inv_std
    shift = beta_ref[...] - mean * scale
    y = jnp.dot(w_ref[...], x_ref[0], preferred_element_type=jnp.float32)
    o_ref[0] = (y * scale + shift).astype(o_ref.dtype)


def kernel(x, w, b, gamma, beta):
    del b
    N, C_in, L = x.shape
    C_out = w.shape[0]

    if L <= L_TILE_MAX:
        l_tile, n_l, L_p = L, 1, L
        x_p = x
    else:
        n_l = pl.cdiv(L, L_TILE_MAX)
        l_tile = _round_up(pl.cdiv(L, n_l), 128)
        L_p = n_l * l_tile
        x_p = jnp.pad(x, ((0, 0), (0, 0), (0, L_p - L))) if L_p != L else x

    gamma2 = gamma.reshape(C_out, 1)
    beta2 = beta.reshape(C_out, 1)
    inv_n = 1.0 / float(N * L)

    part = pl.pallas_call(
        _stats_kernel,
        out_shape=jax.ShapeDtypeStruct((N, n_l, C_out, 2), jnp.float32),
        grid=(N, n_l),
        in_specs=[
            pl.BlockSpec((1, C_in, l_tile), lambda n, l: (n, 0, l)),
            pl.BlockSpec((C_out, C_in), lambda n, l: (0, 0)),
        ],
        out_specs=pl.BlockSpec((1, 1, C_out, 2), lambda n, l: (n, l, 0, 0)),
        compiler_params=pltpu.CompilerParams(
            dimension_semantics=("parallel", "parallel"),
            vmem_limit_bytes=VMEM_LIMIT_BYTES,
        ),
    )(x_p, w)

    out_p = pl.pallas_call(
        functools.partial(_apply_kernel, inv_n=inv_n),
        out_shape=jax.ShapeDtypeStruct((N, C_out, L_p), x.dtype),
        grid=(N, n_l),
        in_specs=[
            pl.BlockSpec((1, C_in, l_tile), lambda n, l: (n, 0, l)),
            pl.BlockSpec((C_out, C_in), lambda n, l: (0, 0)),
            pl.BlockSpec((C_out, 1), lambda n, l: (0, 0)),
            pl.BlockSpec((C_out, 1), lambda n, l: (0, 0)),
            pl.BlockSpec((N, n_l, C_out, 2), lambda n, l: (0, 0, 0, 0)),
        ],
        out_specs=pl.BlockSpec((1, C_out, l_tile), lambda n, l: (n, 0, l)),
        compiler_params=pltpu.CompilerParams(
            dimension_semantics=("parallel", "parallel"),
            vmem_limit_bytes=VMEM_LIMIT_BYTES,
        ),
    )(x_p, w, gamma2, beta2, part)

    return out_p if L_p == L else out_p[:, :, :L]


# speedup vs baseline: 1.4169x; 1.4169x over previous
import functools

import jax
import jax.numpy as jnp
from jax.experimental import pallas as pl
from jax.experimental.pallas import tpu as pltpu

EPS = 1e-5
VMEM_LIMIT_BYTES = 48 * 1024 * 1024
L_TILE_MAX = 2048
FUSED_SCRATCH_LIMIT = 40 * 1024 * 1024


def _round_up(x, m):
    return (x + m - 1) // m * m


def _fused_kernel(x_ref, w_ref, g_ref, b_ref, o_ref,
                  y_scr, sum_scr, sumsq_scr, ss_scr, *, M, inv_n):
    i = pl.program_id(0)

    @pl.when(i == 0)
    def _():
        sum_scr[...] = jnp.zeros_like(sum_scr)
        sumsq_scr[...] = jnp.zeros_like(sumsq_scr)

    @pl.when(i < M)
    def _():
        y = jnp.dot(w_ref[...], x_ref[0], preferred_element_type=jnp.float32)
        y_scr[pl.ds(i, 1), :, :] = y[None]
        sum_scr[...] += y
        sumsq_scr[...] += y * y

    @pl.when(i == M - 1)
    def _():
        s = jnp.sum(sum_scr[...], axis=1, keepdims=True)
        sq = jnp.sum(sumsq_scr[...], axis=1, keepdims=True)
        mean = s * inv_n
        var = jnp.maximum(sq * inv_n - mean * mean, 0.0)
        inv_std = jax.lax.rsqrt(var + EPS)
        scale = g_ref[...] * inv_std
        shift = b_ref[...] - mean * scale
        ss_scr[:, 0:1] = scale
        ss_scr[:, 1:2] = shift

    @pl.when(i >= M)
    def _():
        t = i - M
        ss = ss_scr[...]
        y = y_scr[pl.ds(t, 1), :, :][0]
        o_ref[0] = (y * ss[:, 0:1] + ss[:, 1:2]).astype(o_ref.dtype)


def _stats_kernel(x_ref, w_ref, part_ref):
    y = jnp.dot(w_ref[...], x_ref[0], preferred_element_type=jnp.float32)
    part_ref[0, 0, :, 0:1] = jnp.sum(y, axis=1, keepdims=True)
    part_ref[0, 0, :, 1:2] = jnp.sum(y * y, axis=1, keepdims=True)


def _apply_kernel(x_ref, w_ref, gamma_ref, beta_ref, part_ref, o_ref, *, inv_n):
    part = part_ref[...]
    s = jnp.sum(part[:, :, :, 0:1], axis=(0, 1))
    sq = jnp.sum(part[:, :, :, 1:2], axis=(0, 1))
    mean = s * inv_n
    var = jnp.maximum(sq * inv_n - mean * mean, 0.0)
    inv_std = jax.lax.rsqrt(var + EPS)
    scale = gamma_ref[...] * inv_std
    shift = beta_ref[...] - mean * scale
    y = jnp.dot(w_ref[...], x_ref[0], preferred_element_type=jnp.float32)
    o_ref[0] = (y * scale + shift).astype(o_ref.dtype)


def _two_pass(x_p, w, gamma2, beta2, N, C_in, C_out, n_l, l_tile, L_p, inv_n, out_dtype):
    part = pl.pallas_call(
        _stats_kernel,
        out_shape=jax.ShapeDtypeStruct((N, n_l, C_out, 2), jnp.float32),
        grid=(N, n_l),
        in_specs=[
            pl.BlockSpec((1, C_in, l_tile), lambda n, l: (n, 0, l)),
            pl.BlockSpec((C_out, C_in), lambda n, l: (0, 0)),
        ],
        out_specs=pl.BlockSpec((1, 1, C_out, 2), lambda n, l: (n, l, 0, 0)),
        compiler_params=pltpu.CompilerParams(
            dimension_semantics=("parallel", "parallel"),
            vmem_limit_bytes=VMEM_LIMIT_BYTES,
        ),
    )(x_p, w)

    return pl.pallas_call(
        functools.partial(_apply_kernel, inv_n=inv_n),
        out_shape=jax.ShapeDtypeStruct((N, C_out, L_p), out_dtype),
        grid=(N, n_l),
        in_specs=[
            pl.BlockSpec((1, C_in, l_tile), lambda n, l: (n, 0, l)),
            pl.BlockSpec((C_out, C_in), lambda n, l: (0, 0)),
            pl.BlockSpec((C_out, 1), lambda n, l: (0, 0)),
            pl.BlockSpec((C_out, 1), lambda n, l: (0, 0)),
            pl.BlockSpec((N, n_l, C_out, 2), lambda n, l: (0, 0, 0, 0)),
        ],
        out_specs=pl.BlockSpec((1, C_out, l_tile), lambda n, l: (n, 0, l)),
        compiler_params=pltpu.CompilerParams(
            dimension_semantics=("parallel", "parallel"),
            vmem_limit_bytes=VMEM_LIMIT_BYTES,
        ),
    )(x_p, w, gamma2, beta2, part)


def kernel(x, w, b, gamma, beta):
    del b
    N, C_in, L = x.shape
    C_out = w.shape[0]

    if L <= L_TILE_MAX:
        l_tile, n_l, L_p = L, 1, L
        x_p = x
    else:
        n_l = pl.cdiv(L, L_TILE_MAX)
        l_tile = _round_up(pl.cdiv(L, n_l), 128)
        L_p = n_l * l_tile
        x_p = jnp.pad(x, ((0, 0), (0, 0), (0, L_p - L))) if L_p != L else x

    gamma2 = gamma.reshape(C_out, 1)
    beta2 = beta.reshape(C_out, 1)
    inv_n = 1.0 / float(N * L)
    M = N * n_l

    fused_bytes = 4 * (M * C_out * l_tile + 2 * C_out * l_tile
                       + 2 * C_in * l_tile + 2 * C_out * l_tile)
    if fused_bytes > FUSED_SCRATCH_LIMIT:
        out_p = _two_pass(x_p, w, gamma2, beta2, N, C_in, C_out,
                          n_l, l_tile, L_p, inv_n, x.dtype)
        return out_p if L_p == L else out_p[:, :, :L]

    def x_map(i):
        t = jnp.minimum(i, M - 1)
        return (t // n_l, 0, t % n_l)

    def o_map(i):
        t = jnp.maximum(i - M, 0)
        return (t // n_l, 0, t % n_l)

    out_p = pl.pallas_call(
        functools.partial(_fused_kernel, M=M, inv_n=inv_n),
        out_shape=jax.ShapeDtypeStruct((N, C_out, L_p), x.dtype),
        grid=(2 * M,),
        in_specs=[
            pl.BlockSpec((1, C_in, l_tile), x_map),
            pl.BlockSpec((C_out, C_in), lambda i: (0, 0)),
            pl.BlockSpec((C_out, 1), lambda i: (0, 0)),
            pl.BlockSpec((C_out, 1), lambda i: (0, 0)),
        ],
        out_specs=pl.BlockSpec((1, C_out, l_tile), o_map),
        scratch_shapes=[
            pltpu.VMEM((M, C_out, l_tile), jnp.float32),
            pltpu.VMEM((C_out, l_tile), jnp.float32),
            pltpu.VMEM((C_out, l_tile), jnp.float32),
            pltpu.VMEM((C_out, 2), jnp.float32),
        ],
        compiler_params=pltpu.CompilerParams(
            dimension_semantics=("arbitrary",),
            vmem_limit_bytes=VMEM_LIMIT_BYTES,
        ),
    )(x_p, w, gamma2, beta2)

    return out_p if L_p == L else out_p[:, :, :L]


# speedup vs baseline: 2.3698x; 1.6726x over previous
import functools

import jax
import jax.numpy as jnp
from jax.experimental import pallas as pl
from jax.experimental.pallas import tpu as pltpu

EPS = 1e-5
VMEM_LIMIT_BYTES = 48 * 1024 * 1024
FUSED_VMEM_LIMIT_BYTES = 56 * 1024 * 1024
L_TILE_MAX = 2048
FUSED_BUDGET = 52 * 1024 * 1024


def _round_up(x, m):
    return (x + m - 1) // m * m


def _fused_kernel(x_ref, w_ref, g_ref, b_ref, o_ref,
                  y_scr, sum_scr, sumsq_scr, ss_scr, *, Mb, B, inv_n):
    i = pl.program_id(0)

    @pl.when(i == 0)
    def _():
        sum_scr[...] = jnp.zeros_like(sum_scr)
        sumsq_scr[...] = jnp.zeros_like(sumsq_scr)

    @pl.when(i < Mb)
    def _():
        for b in range(B):
            y = jnp.dot(w_ref[...], x_ref[b], preferred_element_type=jnp.float32)
            y_scr[pl.ds(i * B + b, 1), :, :] = y[None]
            sum_scr[...] += y
            sumsq_scr[...] += y * y

    @pl.when(i == Mb - 1)
    def _():
        s = jnp.sum(sum_scr[...], axis=1, keepdims=True)
        sq = jnp.sum(sumsq_scr[...], axis=1, keepdims=True)
        mean = s * inv_n
        var = jnp.maximum(sq * inv_n - mean * mean, 0.0)
        inv_std = jax.lax.rsqrt(var + EPS)
        scale = g_ref[...] * inv_std
        shift = b_ref[...] - mean * scale
        ss_scr[:, 0:1] = scale
        ss_scr[:, 1:2] = shift

    @pl.when(i >= Mb)
    def _():
        t = i - Mb
        ss = ss_scr[...]
        scale, shift = ss[:, 0:1], ss[:, 1:2]
        for b in range(B):
            y = y_scr[pl.ds(t * B + b, 1), :, :][0]
            o_ref[b] = (y * scale + shift).astype(o_ref.dtype)


def _stats_kernel(x_ref, w_ref, part_ref):
    y = jnp.dot(w_ref[...], x_ref[0], preferred_element_type=jnp.float32)
    part_ref[0, 0, :, 0:1] = jnp.sum(y, axis=1, keepdims=True)
    part_ref[0, 0, :, 1:2] = jnp.sum(y * y, axis=1, keepdims=True)


def _apply_kernel(x_ref, w_ref, gamma_ref, beta_ref, part_ref, o_ref, *, inv_n):
    part = part_ref[...]
    s = jnp.sum(part[:, :, :, 0:1], axis=(0, 1))
    sq = jnp.sum(part[:, :, :, 1:2], axis=(0, 1))
    mean = s * inv_n
    var = jnp.maximum(sq * inv_n - mean * mean, 0.0)
    inv_std = jax.lax.rsqrt(var + EPS)
    scale = gamma_ref[...] * inv_std
    shift = beta_ref[...] - mean * scale
    y = jnp.dot(w_ref[...], x_ref[0], preferred_element_type=jnp.float32)
    o_ref[0] = (y * scale + shift).astype(o_ref.dtype)


def _two_pass(x_p, w, gamma2, beta2, N, C_in, C_out, n_l, l_tile, L_p, inv_n, out_dtype):
    part = pl.pallas_call(
        _stats_kernel,
        out_shape=jax.ShapeDtypeStruct((N, n_l, C_out, 2), jnp.float32),
        grid=(N, n_l),
        in_specs=[
            pl.BlockSpec((1, C_in, l_tile), lambda n, l: (n, 0, l)),
            pl.BlockSpec((C_out, C_in), lambda n, l: (0, 0)),
        ],
        out_specs=pl.BlockSpec((1, 1, C_out, 2), lambda n, l: (n, l, 0, 0)),
        compiler_params=pltpu.CompilerParams(
            dimension_semantics=("parallel", "parallel"),
            vmem_limit_bytes=VMEM_LIMIT_BYTES,
        ),
    )(x_p, w)

    return pl.pallas_call(
        functools.partial(_apply_kernel, inv_n=inv_n),
        out_shape=jax.ShapeDtypeStruct((N, C_out, L_p), out_dtype),
        grid=(N, n_l),
        in_specs=[
            pl.BlockSpec((1, C_in, l_tile), lambda n, l: (n, 0, l)),
            pl.BlockSpec((C_out, C_in), lambda n, l: (0, 0)),
            pl.BlockSpec((C_out, 1), lambda n, l: (0, 0)),
            pl.BlockSpec((C_out, 1), lambda n, l: (0, 0)),
            pl.BlockSpec((N, n_l, C_out, 2), lambda n, l: (0, 0, 0, 0)),
        ],
        out_specs=pl.BlockSpec((1, C_out, l_tile), lambda n, l: (n, 0, l)),
        compiler_params=pltpu.CompilerParams(
            dimension_semantics=("parallel", "parallel"),
            vmem_limit_bytes=VMEM_LIMIT_BYTES,
        ),
    )(x_p, w, gamma2, beta2, part)


def kernel(x, w, b, gamma, beta):
    del b
    N, C_in, L = x.shape
    C_out = w.shape[0]

    if L <= L_TILE_MAX:
        l_tile, n_l, L_p = L, 1, L
        x_p = x
    else:
        n_l = pl.cdiv(L, L_TILE_MAX)
        l_tile = _round_up(pl.cdiv(L, n_l), 128)
        L_p = n_l * l_tile
        x_p = jnp.pad(x, ((0, 0), (0, 0), (0, L_p - L))) if L_p != L else x

    gamma2 = gamma.reshape(C_out, 1)
    beta2 = beta.reshape(C_out, 1)
    inv_n = 1.0 / float(N * L)
    M = N * n_l

    B = 0
    if n_l == 1:
        for cand in (8, 4, 2, 1):
            if M % cand:
                continue
            need = 4 * (M * C_out * l_tile + 2 * C_out * l_tile
                        + 2 * cand * C_in * l_tile + 2 * cand * C_out * l_tile)
            if need <= FUSED_BUDGET:
                B = cand
                break
    if B == 0:
        out_p = _two_pass(x_p, w, gamma2, beta2, N, C_in, C_out,
                          n_l, l_tile, L_p, inv_n, x.dtype)
        return out_p if L_p == L else out_p[:, :, :L]

    Mb = M // B

    def x_map(i):
        return (jnp.minimum(i, Mb - 1), 0, 0)

    def o_map(i):
        return (jnp.maximum(i - Mb, 0), 0, 0)

    out_p = pl.pallas_call(
        functools.partial(_fused_kernel, Mb=Mb, B=B, inv_n=inv_n),
        out_shape=jax.ShapeDtypeStruct((N, C_out, L_p), x.dtype),
        grid=(2 * Mb,),
        in_specs=[
            pl.BlockSpec((B, C_in, l_tile), x_map),
            pl.BlockSpec((C_out, C_in), lambda i: (0, 0)),
            pl.BlockSpec((C_out, 1), lambda i: (0, 0)),
            pl.BlockSpec((C_out, 1), lambda i: (0, 0)),
        ],
        out_specs=pl.BlockSpec((B, C_out, l_tile), o_map),
        scratch_shapes=[
            pltpu.VMEM((M, C_out, l_tile), jnp.float32),
            pltpu.VMEM((C_out, l_tile), jnp.float32),
            pltpu.VMEM((C_out, l_tile), jnp.float32),
            pltpu.VMEM((C_out, 2), jnp.float32),
        ],
        compiler_params=pltpu.CompilerParams(
            dimension_semantics=("arbitrary",),
            vmem_limit_bytes=FUSED_VMEM_LIMIT_BYTES,
        ),
    )(x_p, w, gamma2, beta2)

    return out_p if L_p == L else out_p[:, :, :L]


# speedup vs baseline: 2.4399x; 1.0296x over previous
import functools

import jax
import jax.numpy as jnp
from jax.experimental import pallas as pl
from jax.experimental.pallas import tpu as pltpu

EPS = 1e-5
VMEM_LIMIT_BYTES = 48 * 1024 * 1024
FUSED_VMEM_LIMIT_BYTES = 56 * 1024 * 1024
L_TILE_MAX = 2048
FUSED_BUDGET = 52 * 1024 * 1024
Y_SCR_DTYPE = jnp.bfloat16
Y_SCR_ITEMSIZE = 2


def _round_up(x, m):
    return (x + m - 1) // m * m


def _fused_kernel(x_ref, w_ref, g_ref, b_ref, o_ref,
                  y_scr, sum_scr, sumsq_scr, ss_scr, *, Mb, B, inv_n):
    i = pl.program_id(0)

    @pl.when(i == 0)
    def _():
        sum_scr[...] = jnp.zeros_like(sum_scr)
        sumsq_scr[...] = jnp.zeros_like(sumsq_scr)

    @pl.when(i < Mb)
    def _():
        for b in range(B):
            y = jnp.dot(w_ref[...], x_ref[b], preferred_element_type=jnp.float32)
            y_scr[pl.ds(i * B + b, 1), :, :] = y[None].astype(y_scr.dtype)
            sum_scr[...] += y
            sumsq_scr[...] += y * y

    @pl.when(i == Mb - 1)
    def _():
        s = jnp.sum(sum_scr[...], axis=1, keepdims=True)
        sq = jnp.sum(sumsq_scr[...], axis=1, keepdims=True)
        mean = s * inv_n
        var = jnp.maximum(sq * inv_n - mean * mean, 0.0)
        inv_std = jax.lax.rsqrt(var + EPS)
        scale = g_ref[...] * inv_std
        shift = b_ref[...] - mean * scale
        ss_scr[:, 0:1] = scale
        ss_scr[:, 1:2] = shift

    @pl.when(i >= Mb)
    def _():
        t = i - Mb
        ss = ss_scr[...]
        scale, shift = ss[:, 0:1], ss[:, 1:2]
        for b in range(B):
            y = y_scr[pl.ds(t * B + b, 1), :, :][0].astype(jnp.float32)
            o_ref[b] = (y * scale + shift).astype(o_ref.dtype)


def _stats_kernel(x_ref, w_ref, part_ref):
    y = jnp.dot(w_ref[...], x_ref[0], preferred_element_type=jnp.float32)
    part_ref[0, 0, :, 0:1] = jnp.sum(y, axis=1, keepdims=True)
    part_ref[0, 0, :, 1:2] = jnp.sum(y * y, axis=1, keepdims=True)


def _apply_kernel(x_ref, w_ref, gamma_ref, beta_ref, part_ref, o_ref, *, inv_n):
    part = part_ref[...]
    s = jnp.sum(part[:, :, :, 0:1], axis=(0, 1))
    sq = jnp.sum(part[:, :, :, 1:2], axis=(0, 1))
    mean = s * inv_n
    var = jnp.maximum(sq * inv_n - mean * mean, 0.0)
    inv_std = jax.lax.rsqrt(var + EPS)
    scale = gamma_ref[...] * inv_std
    shift = beta_ref[...] - mean * scale
    y = jnp.dot(w_ref[...], x_ref[0], preferred_element_type=jnp.float32)
    o_ref[0] = (y * scale + shift).astype(o_ref.dtype)


def _two_pass(x_p, w, gamma2, beta2, N, C_in, C_out, n_l, l_tile, L_p, inv_n, out_dtype):
    part = pl.pallas_call(
        _stats_kernel,
        out_shape=jax.ShapeDtypeStruct((N, n_l, C_out, 2), jnp.float32),
        grid=(N, n_l),
        in_specs=[
            pl.BlockSpec((1, C_in, l_tile), lambda n, l: (n, 0, l)),
            pl.BlockSpec((C_out, C_in), lambda n, l: (0, 0)),
        ],
        out_specs=pl.BlockSpec((1, 1, C_out, 2), lambda n, l: (n, l, 0, 0)),
        compiler_params=pltpu.CompilerParams(
            dimension_semantics=("parallel", "parallel"),
            vmem_limit_bytes=VMEM_LIMIT_BYTES,
        ),
    )(x_p, w)

    return pl.pallas_call(
        functools.partial(_apply_kernel, inv_n=inv_n),
        out_shape=jax.ShapeDtypeStruct((N, C_out, L_p), out_dtype),
        grid=(N, n_l),
        in_specs=[
            pl.BlockSpec((1, C_in, l_tile), lambda n, l: (n, 0, l)),
            pl.BlockSpec((C_out, C_in), lambda n, l: (0, 0)),
            pl.BlockSpec((C_out, 1), lambda n, l: (0, 0)),
            pl.BlockSpec((C_out, 1), lambda n, l: (0, 0)),
            pl.BlockSpec((N, n_l, C_out, 2), lambda n, l: (0, 0, 0, 0)),
        ],
        out_specs=pl.BlockSpec((1, C_out, l_tile), lambda n, l: (n, 0, l)),
        compiler_params=pltpu.CompilerParams(
            dimension_semantics=("parallel", "parallel"),
            vmem_limit_bytes=VMEM_LIMIT_BYTES,
        ),
    )(x_p, w, gamma2, beta2, part)


def kernel(x, w, b, gamma, beta):
    del b
    N, C_in, L = x.shape
    C_out = w.shape[0]

    if L <= L_TILE_MAX:
        l_tile, n_l, L_p = L, 1, L
        x_p = x
    else:
        n_l = pl.cdiv(L, L_TILE_MAX)
        l_tile = _round_up(pl.cdiv(L, n_l), 128)
        L_p = n_l * l_tile
        x_p = jnp.pad(x, ((0, 0), (0, 0), (0, L_p - L))) if L_p != L else x

    gamma2 = gamma.reshape(C_out, 1)
    beta2 = beta.reshape(C_out, 1)
    inv_n = 1.0 / float(N * L)
    M = N * n_l

    B = 0
    if n_l == 1:
        for cand in (8, 4, 2, 1):
            if M % cand:
                continue
            need = (Y_SCR_ITEMSIZE * M * C_out * l_tile
                    + 4 * (2 * C_out * l_tile
                           + 2 * cand * C_in * l_tile + 2 * cand * C_out * l_tile))
            if need <= FUSED_BUDGET:
                B = cand
                break
    if B == 0:
        out_p = _two_pass(x_p, w, gamma2, beta2, N, C_in, C_out,
                          n_l, l_tile, L_p, inv_n, x.dtype)
        return out_p if L_p == L else out_p[:, :, :L]

    Mb = M // B

    def x_map(i):
        return (jnp.minimum(i, Mb - 1), 0, 0)

    def o_map(i):
        return (jnp.maximum(i - Mb, 0), 0, 0)

    out_p = pl.pallas_call(
        functools.partial(_fused_kernel, Mb=Mb, B=B, inv_n=inv_n),
        out_shape=jax.ShapeDtypeStruct((N, C_out, L_p), x.dtype),
        grid=(2 * Mb,),
        in_specs=[
            pl.BlockSpec((B, C_in, l_tile), x_map),
            pl.BlockSpec((C_out, C_in), lambda i: (0, 0)),
            pl.BlockSpec((C_out, 1), lambda i: (0, 0)),
            pl.BlockSpec((C_out, 1), lambda i: (0, 0)),
        ],
        out_specs=pl.BlockSpec((B, C_out, l_tile), o_map),
        scratch_shapes=[
            pltpu.VMEM((M, C_out, l_tile), Y_SCR_DTYPE),
            pltpu.VMEM((C_out, l_tile), jnp.float32),
            pltpu.VMEM((C_out, l_tile), jnp.float32),
            pltpu.VMEM((C_out, 2), jnp.float32),
        ],
        compiler_params=pltpu.CompilerParams(
            dimension_semantics=("arbitrary",),
            vmem_limit_bytes=FUSED_VMEM_LIMIT_BYTES,
        ),
    )(x_p, w, gamma2, beta2)

    return out_p if L_p == L else out_p[:, :, :L]


# speedup vs baseline: 2.4646x; 1.0101x over previous
import functools

import jax
import jax.numpy as jnp
from jax.experimental import pallas as pl
from jax.experimental.pallas import tpu as pltpu

EPS = 1e-5
VMEM_LIMIT_BYTES = 48 * 1024 * 1024
FUSED_VMEM_LIMIT_BYTES = 56 * 1024 * 1024
L_TILE_MAX = 2048
FUSED_BUDGET = 52 * 1024 * 1024
Y_SCR_DTYPE = jnp.bfloat16
Y_SCR_ITEMSIZE = 2


def _round_up(x, m):
    return (x + m - 1) // m * m


def _fused_kernel(x_ref, w_ref, g_ref, b_ref, o_ref,
                  y_scr, sum_scr, sumsq_scr, ss_scr, *, Mi, Bi, Bo, inv_n):
    i = pl.program_id(0)

    @pl.when(i == 0)
    def _():
        sum_scr[...] = jnp.zeros_like(sum_scr)
        sumsq_scr[...] = jnp.zeros_like(sumsq_scr)

    @pl.when(i < Mi)
    def _():
        for b in range(Bi):
            y = jnp.dot(w_ref[...], x_ref[b], preferred_element_type=jnp.float32)
            y_scr[pl.ds(i * Bi + b, 1), :, :] = y[None].astype(y_scr.dtype)
            sum_scr[...] += y
            sumsq_scr[...] += y * y

    @pl.when(i == Mi - 1)
    def _():
        s = jnp.sum(sum_scr[...], axis=1, keepdims=True)
        sq = jnp.sum(sumsq_scr[...], axis=1, keepdims=True)
        mean = s * inv_n
        var = jnp.maximum(sq * inv_n - mean * mean, 0.0)
        inv_std = jax.lax.rsqrt(var + EPS)
        scale = g_ref[...] * inv_std
        shift = b_ref[...] - mean * scale
        ss_scr[:, 0:1] = scale
        ss_scr[:, 1:2] = shift

    @pl.when(i >= Mi)
    def _():
        t = i - Mi
        ss = ss_scr[...]
        scale, shift = ss[:, 0:1], ss[:, 1:2]
        for b in range(Bo):
            y = y_scr[pl.ds(t * Bo + b, 1), :, :][0].astype(jnp.float32)
            o_ref[b] = (y * scale + shift).astype(o_ref.dtype)


def _stats_kernel(x_ref, w_ref, part_ref):
    y = jnp.dot(w_ref[...], x_ref[0], preferred_element_type=jnp.float32)
    part_ref[0, 0, :, 0:1] = jnp.sum(y, axis=1, keepdims=True)
    part_ref[0, 0, :, 1:2] = jnp.sum(y * y, axis=1, keepdims=True)


def _apply_kernel(x_ref, w_ref, gamma_ref, beta_ref, part_ref, o_ref, *, inv_n):
    part = part_ref[...]
    s = jnp.sum(part[:, :, :, 0:1], axis=(0, 1))
    sq = jnp.sum(part[:, :, :, 1:2], axis=(0, 1))
    mean = s * inv_n
    var = jnp.maximum(sq * inv_n - mean * mean, 0.0)
    inv_std = jax.lax.rsqrt(var + EPS)
    scale = gamma_ref[...] * inv_std
    shift = beta_ref[...] - mean * scale
    y = jnp.dot(w_ref[...], x_ref[0], preferred_element_type=jnp.float32)
    o_ref[0] = (y * scale + shift).astype(o_ref.dtype)


def _two_pass(x_p, w, gamma2, beta2, N, C_in, C_out, n_l, l_tile, L_p, inv_n, out_dtype):
    part = pl.pallas_call(
        _stats_kernel,
        out_shape=jax.ShapeDtypeStruct((N, n_l, C_out, 2), jnp.float32),
        grid=(N, n_l),
        in_specs=[
            pl.BlockSpec((1, C_in, l_tile), lambda n, l: (n, 0, l)),
            pl.BlockSpec((C_out, C_in), lambda n, l: (0, 0)),
        ],
        out_specs=pl.BlockSpec((1, 1, C_out, 2), lambda n, l: (n, l, 0, 0)),
        compiler_params=pltpu.CompilerParams(
            dimension_semantics=("parallel", "parallel"),
            vmem_limit_bytes=VMEM_LIMIT_BYTES,
        ),
    )(x_p, w)

    return pl.pallas_call(
        functools.partial(_apply_kernel, inv_n=inv_n),
        out_shape=jax.ShapeDtypeStruct((N, C_out, L_p), out_dtype),
        grid=(N, n_l),
        in_specs=[
            pl.BlockSpec((1, C_in, l_tile), lambda n, l: (n, 0, l)),
            pl.BlockSpec((C_out, C_in), lambda n, l: (0, 0)),
            pl.BlockSpec((C_out, 1), lambda n, l: (0, 0)),
            pl.BlockSpec((C_out, 1), lambda n, l: (0, 0)),
            pl.BlockSpec((N, n_l, C_out, 2), lambda n, l: (0, 0, 0, 0)),
        ],
        out_specs=pl.BlockSpec((1, C_out, l_tile), lambda n, l: (n, 0, l)),
        compiler_params=pltpu.CompilerParams(
            dimension_semantics=("parallel", "parallel"),
            vmem_limit_bytes=VMEM_LIMIT_BYTES,
        ),
    )(x_p, w, gamma2, beta2, part)


def kernel(x, w, b, gamma, beta):
    del b
    N, C_in, L = x.shape
    C_out = w.shape[0]

    if L <= L_TILE_MAX:
        l_tile, n_l, L_p = L, 1, L
        x_p = x
    else:
        n_l = pl.cdiv(L, L_TILE_MAX)
        l_tile = _round_up(pl.cdiv(L, n_l), 128)
        L_p = n_l * l_tile
        x_p = jnp.pad(x, ((0, 0), (0, 0), (0, L_p - L))) if L_p != L else x

    gamma2 = gamma.reshape(C_out, 1)
    beta2 = beta.reshape(C_out, 1)
    inv_n = 1.0 / float(N * L)
    M = N * n_l

    Bi = 0
    if n_l == 1:
        for cand in (8, 4, 2, 1):
            if M % cand:
                continue
            need = (Y_SCR_ITEMSIZE * M * C_out * l_tile
                    + 4 * (2 * C_out * l_tile
                           + 2 * cand * C_in * l_tile + 2 * cand * C_out * l_tile))
            if need <= FUSED_BUDGET:
                Bi = cand
                break
    if Bi == 0:
        out_p = _two_pass(x_p, w, gamma2, beta2, N, C_in, C_out,
                          n_l, l_tile, L_p, inv_n, x.dtype)
        return out_p if L_p == L else out_p[:, :, :L]

    Bo = Bi // 2 if (Bi > 1 and M % (Bi // 2) == 0) else Bi
    Mi, Mo = M // Bi, M // Bo

    def x_map(i):
        return (jnp.minimum(i, Mi - 1), 0, 0)

    def o_map(i):
        return (jnp.maximum(i - Mi, 0), 0, 0)

    out_p = pl.pallas_call(
        functools.partial(_fused_kernel, Mi=Mi, Bi=Bi, Bo=Bo, inv_n=inv_n),
        out_shape=jax.ShapeDtypeStruct((N, C_out, L_p), x.dtype),
        grid=(Mi + Mo,),
        in_specs=[
            pl.BlockSpec((Bi, C_in, l_tile), x_map),
            pl.BlockSpec((C_out, C_in), lambda i: (0, 0)),
            pl.BlockSpec((C_out, 1), lambda i: (0, 0)),
            pl.BlockSpec((C_out, 1), lambda i: (0, 0)),
        ],
        out_specs=pl.BlockSpec((Bo, C_out, l_tile), o_map),
        scratch_shapes=[
            pltpu.VMEM((M, C_out, l_tile), Y_SCR_DTYPE),
            pltpu.VMEM((C_out, l_tile), jnp.float32),
            pltpu.VMEM((C_out, l_tile), jnp.float32),
            pltpu.VMEM((C_out, 2), jnp.float32),
        ],
        compiler_params=pltpu.CompilerParams(
            dimension_semantics=("arbitrary",),
            vmem_limit_bytes=FUSED_VMEM_LIMIT_BYTES,
        ),
    )(x_p, w, gamma2, beta2)

    return out_p if L_p == L else out_p[:, :, :L]


# speedup vs baseline: 2.5422x; 1.0315x over previous
import functools

import jax
import jax.numpy as jnp
from jax.experimental import pallas as pl
from jax.experimental.pallas import tpu as pltpu

EPS = 1e-5
VMEM_LIMIT_BYTES = 48 * 1024 * 1024
L_TILE_MAX = 2048
RESIDENT_BUDGET = 44 * 1024 * 1024


def _round_up(x, m):
    return (x + m - 1) // m * m


def _fused_kernel(x_hbm, w_ref, g_ref, b_ref, o_ref,
                  x_scr, gram_scr, s_scr, ss_scr, sems,
                  *, NC, Ci, Bo, inv_n):
    i = pl.program_id(0)

    @pl.when(i == 0)
    def _():
        gram_scr[...] = jnp.zeros_like(gram_scr)
        s_scr[...] = jnp.zeros_like(s_scr)
        for k in range(NC):
            pltpu.make_async_copy(
                x_hbm.at[pl.ds(k * Ci, Ci)],
                x_scr.at[pl.ds(k * Ci, Ci)],
                sems.at[k],
            ).start()

    @pl.when(i < NC)
    def _():
        pltpu.make_async_copy(
            x_hbm.at[pl.ds(i * Ci, Ci)],
            x_scr.at[pl.ds(i * Ci, Ci)],
            sems.at[i],
        ).wait()
        for b in range(Ci):
            xb = x_scr[pl.ds(i * Ci + b, 1), :, :][0]
            gram_scr[...] += jax.lax.dot_general(
                xb, xb, (((1,), (1,)), ((), ())),
                preferred_element_type=jnp.float32)
            s_scr[...] += jnp.sum(xb, axis=1, keepdims=True)

    @pl.when(i == NC - 1)
    def _():
        w_mat = w_ref[...]
        mean = jnp.dot(w_mat, s_scr[...],
                       preferred_element_type=jnp.float32) * inv_n
        t = jnp.dot(w_mat, gram_scr[...],
                    preferred_element_type=jnp.float32)
        sumsq = jnp.sum(t * w_mat, axis=1, keepdims=True)
        var = jnp.maximum(sumsq * inv_n - mean * mean, 0.0)
        inv_std = jax.lax.rsqrt(var + EPS)
        scale = g_ref[...] * inv_std
        shift = b_ref[...] - mean * scale
        ss_scr[:, 0:1] = scale
        ss_scr[:, 1:2] = shift

    @pl.when(i >= NC)
    def _():
        t = i - NC
        ss = ss_scr[...]
        scale, shift = ss[:, 0:1], ss[:, 1:2]
        for b in range(Bo):
            xb = x_scr[pl.ds(t * Bo + b, 1), :, :][0]
            y = jnp.dot(w_ref[...], xb, preferred_element_type=jnp.float32)
            o_ref[b] = (y * scale + shift).astype(o_ref.dtype)


def _stats_kernel(x_ref, w_ref, part_ref):
    y = jnp.dot(w_ref[...], x_ref[0], preferred_element_type=jnp.float32)
    part_ref[0, 0, :, 0:1] = jnp.sum(y, axis=1, keepdims=True)
    part_ref[0, 0, :, 1:2] = jnp.sum(y * y, axis=1, keepdims=True)


def _apply_kernel(x_ref, w_ref, gamma_ref, beta_ref, part_ref, o_ref, *, inv_n):
    part = part_ref[...]
    s = jnp.sum(part[:, :, :, 0:1], axis=(0, 1))
    sq = jnp.sum(part[:, :, :, 1:2], axis=(0, 1))
    mean = s * inv_n
    var = jnp.maximum(sq * inv_n - mean * mean, 0.0)
    inv_std = jax.lax.rsqrt(var + EPS)
    scale = gamma_ref[...] * inv_std
    shift = beta_ref[...] - mean * scale
    y = jnp.dot(w_ref[...], x_ref[0], preferred_element_type=jnp.float32)
    o_ref[0] = (y * scale + shift).astype(o_ref.dtype)


def _two_pass(x_p, w, gamma2, beta2, N, C_in, C_out, n_l, l_tile, L_p, inv_n, out_dtype):
    part = pl.pallas_call(
        _stats_kernel,
        out_shape=jax.ShapeDtypeStruct((N, n_l, C_out, 2), jnp.float32),
        grid=(N, n_l),
        in_specs=[
            pl.BlockSpec((1, C_in, l_tile), lambda n, l: (n, 0, l)),
            pl.BlockSpec((C_out, C_in), lambda n, l: (0, 0)),
        ],
        out_specs=pl.BlockSpec((1, 1, C_out, 2), lambda n, l: (n, l, 0, 0)),
        compiler_params=pltpu.CompilerParams(
            dimension_semantics=("parallel", "parallel"),
            vmem_limit_bytes=VMEM_LIMIT_BYTES,
        ),
    )(x_p, w)

    return pl.pallas_call(
        functools.partial(_apply_kernel, inv_n=inv_n),
        out_shape=jax.ShapeDtypeStruct((N, C_out, L_p), out_dtype),
        grid=(N, n_l),
        in_specs=[
            pl.BlockSpec((1, C_in, l_tile), lambda n, l: (n, 0, l)),
            pl.BlockSpec((C_out, C_in), lambda n, l: (0, 0)),
            pl.BlockSpec((C_out, 1), lambda n, l: (0, 0)),
            pl.BlockSpec((C_out, 1), lambda n, l: (0, 0)),
            pl.BlockSpec((N, n_l, C_out, 2), lambda n, l: (0, 0, 0, 0)),
        ],
        out_specs=pl.BlockSpec((1, C_out, l_tile), lambda n, l: (n, 0, l)),
        compiler_params=pltpu.CompilerParams(
            dimension_semantics=("parallel", "parallel"),
            vmem_limit_bytes=VMEM_LIMIT_BYTES,
        ),
    )(x_p, w, gamma2, beta2, part)


def kernel(x, w, b, gamma, beta):
    del b
    N, C_in, L = x.shape
    C_out = w.shape[0]

    if L <= L_TILE_MAX:
        l_tile, n_l, L_p = L, 1, L
        x_p = x
    else:
        n_l = pl.cdiv(L, L_TILE_MAX)
        l_tile = _round_up(pl.cdiv(L, n_l), 128)
        L_p = n_l * l_tile
        x_p = jnp.pad(x, ((0, 0), (0, 0), (0, L_p - L))) if L_p != L else x

    gamma2 = gamma.reshape(C_out, 1)
    beta2 = beta.reshape(C_out, 1)
    inv_n = 1.0 / float(N * L)

    Ci = Bo = 0
    if n_l == 1:
        for cand in (4, 2, 1):
            if N % cand:
                continue
            need = 4 * (N * C_in * l_tile + 2 * cand * C_out * l_tile
                        + C_in * C_in + 2 * C_in)
            if need <= RESIDENT_BUDGET:
                Ci = Bo = cand
                break
    if Ci == 0:
        out_p = _two_pass(x_p, w, gamma2, beta2, N, C_in, C_out,
                          n_l, l_tile, L_p, inv_n, x.dtype)
        return out_p if L_p == L else out_p[:, :, :L]

    NC, Mo = N // Ci, N // Bo

    def o_map(i):
        return (jnp.maximum(i - NC, 0), 0, 0)

    out_p = pl.pallas_call(
        functools.partial(_fused_kernel, NC=NC, Ci=Ci, Bo=Bo, inv_n=inv_n),
        out_shape=jax.ShapeDtypeStruct((N, C_out, L_p), x.dtype),
        grid=(NC + Mo,),
        in_specs=[
            pl.BlockSpec(memory_space=pl.ANY),
            pl.BlockSpec((C_out, C_in), lambda i: (0, 0)),
            pl.BlockSpec((C_out, 1), lambda i: (0, 0)),
            pl.BlockSpec((C_out, 1), lambda i: (0, 0)),
        ],
        out_specs=pl.BlockSpec((Bo, C_out, l_tile), o_map),
        scratch_shapes=[
            pltpu.VMEM((N, C_in, l_tile), jnp.float32),
            pltpu.VMEM((C_in, C_in), jnp.float32),
            pltpu.VMEM((C_in, 1), jnp.float32),
            pltpu.VMEM((C_out, 2), jnp.float32),
            pltpu.SemaphoreType.DMA((NC,)),
        ],
        compiler_params=pltpu.CompilerParams(
            dimension_semantics=("arbitrary",),
            vmem_limit_bytes=VMEM_LIMIT_BYTES,
        ),
    )(x_p, w, gamma2, beta2)

    return out_p if L_p == L else out_p[:, :, :L]
